# baseline (speedup 1.0000x reference)
import functools

import jax
import jax.numpy as jnp
from jax import lax
from jax.experimental import pallas as pl
from jax.experimental.pallas import tpu as pltpu
from jax.experimental.pallas import tpu_sc as plsc

N = 10000
E = 320000
C_IN = 128
HID = 256
C_OUT = 128

NC = 2
NS = 16
NW = NC * NS

NPW = 320
NPAD = NW * NPW
RPT_LEN = NPW + 24
RPT_PAD = (NW - 1) * NPW + RPT_LEN
CH = 64
FB = 16
EPAD = E + CH

GA = -2.0 * 0.7978845608028654
GB = GA * 0.044715

VB = HID // 16


def _mm_ps_body(x_ref, w_ref, b1_ref, p_ref, s_ref):
    ps = jnp.dot(x_ref[...], w_ref[...], preferred_element_type=jnp.float32)
    p_ref[...] = ps[:, :HID]
    s_ref[...] = ps[:, HID:] + b1_ref[...]


def _mm_out_body(g_ref, w2_ref, b2_ref, rhi_ref, rlo_ref, o_ref):
    cnt = (rhi_ref[0, 0, :] - rlo_ref[0, 0, :]).astype(jnp.float32)
    scale = 1.0 / jnp.maximum(cnt, 1.0)
    gs = g_ref[...] * scale[:, None]
    y = jnp.dot(gs, w2_ref[...], preferred_element_type=jnp.float32)
    o_ref[...] = y + b2_ref[...] * (cnt > 0.0).astype(jnp.float32)[:, None]


def _sc_segment_gelu(p_hbm, s_hbm, idx_hbm, rpt_hbm, g_hbm,
                     rpt_v, idx2_v, rows2_v, s_all, flush_v,
                     semi, semg, sem2):
    c = lax.axis_index("c")
    s = lax.axis_index("s")
    wid = s * NC + c
    n0 = wid * NPW

    pltpu.async_copy(rpt_hbm.at[pl.ds(pl.multiple_of(n0, 8), RPT_LEN)],
                     rpt_v, sem2).wait()
    pltpu.async_copy(s_hbm.at[pl.ds(pl.multiple_of(n0, 8), NPW)],
                     s_all, sem2).wait()

    def rv(k):
        return rpt_v[pl.ds(k, 16)][0]

    zeros16 = jnp.zeros((16,), jnp.float32)

    def issue_idx(a, slot):
        pltpu.async_copy(
            idx_hbm.at[pl.ds(pl.multiple_of(jnp.minimum(a, E), 8), CH)],
            idx2_v.at[slot], semi)

    def issue_gather(slot):
        pltpu.async_copy(p_hbm.at[idx2_v.at[slot]], rows2_v.at[slot], semg)

    def drain_gather():
        pltpu.make_async_copy(p_hbm.at[pl.ds(0, CH)], rows2_v.at[0],
                              semg).wait()

    def drain_idx():
        pltpu.make_async_copy(idx_hbm.at[pl.ds(0, CH)], idx2_v.at[0],
                              semi).wait()

    e_start = rv(0)
    e_end = rv(NPW)
    a_init = (e_start // 8) * 8
    nchunks = (e_end - a_init + CH - 1) // CH
    n_events = NPW + jnp.maximum(nchunks - 1, 0)

    issue_idx(a_init, 0)
    drain_idx()
    issue_gather(0)
    issue_idx(a_init + CH, 1)
    drain_gather()
    drain_idx()
    issue_gather(1)
    issue_idx(a_init + 2 * CH, 0)

    def zero_flush():
        for b in range(FB):
            for j in range(VB):
                flush_v[b, pl.ds(j * 16, 16)] = zeros16

    zero_flush()

    def event_body(_, st):
        i, a, p, par = st
        e1 = rv(i + 1)
        chunk_end = a + CH
        hi = jnp.minimum(e1, chunk_end)
        srow = tuple(s_all[i, pl.ds(j * 16, 16)] for j in range(VB))
        sl = i % FB

        def edge_body(r, carry):
            for j in range(VB):
                x = rows2_v[par, r, pl.ds(j * 16, 16)] + srow[j]
                z = x * (GA + GB * (x * x))
                plsc.addupdate(flush_v.at[sl, pl.ds(j * 16, 16)],
                               x / (1.0 + jnp.exp(z)))
            return carry

        lax.fori_loop(p - a, hi - a, edge_body, 0)
        node_done = jnp.logical_and(e1 <= chunk_end, i < NPW)

        @pl.when(jnp.logical_and(node_done, sl == FB - 1))
        def _():
            pltpu.async_copy(
                flush_v,
                g_hbm.at[pl.ds(pl.multiple_of(n0 + (i // FB) * FB, 8), FB)],
                sem2).wait()
            zero_flush()

        npar = 1 - par

        @pl.when(jnp.logical_not(node_done))
        def _():
            drain_gather()
            drain_idx()
            issue_gather(par)
            issue_idx(a + 3 * CH, npar)

        i = i + node_done.astype(jnp.int32)
        a = jnp.where(node_done, a, a + CH)
        par = jnp.where(node_done, par, npar)
        return (i, a, hi, par)

    lax.fori_loop(0, n_events, event_body, (0, a_init, e_start, 0))

    drain_gather()
    drain_idx()


@functools.partial(
    pl.kernel,
    mesh=plsc.VectorSubcoreMesh(core_axis_name="c", subcore_axis_name="s"),
    out_type=jax.ShapeDtypeStruct((NPAD, HID), jnp.float32),
    scratch_types=[
        pltpu.VMEM((RPT_LEN,), jnp.int32),
        pltpu.VMEM((2, CH), jnp.int32),
        pltpu.VMEM((2, CH, HID), jnp.float32),
        pltpu.VMEM((NPW, HID), jnp.float32),
        pltpu.VMEM((FB, HID), jnp.float32),
        pltpu.SemaphoreType.DMA,
        pltpu.SemaphoreType.DMA,
        pltpu.SemaphoreType.DMA,
    ],
)
def _sc_kernel(p_hbm, s_hbm, idx_hbm, rpt_hbm, g_hbm,
               rpt_v, idx2_v, rows2_v, s_all, flush_v, semi, semg, sem2):
    _sc_segment_gelu(p_hbm, s_hbm, idx_hbm, rpt_hbm, g_hbm,
                     rpt_v, idx2_v, rows2_v, s_all, flush_v,
                     semi, semg, sem2)


def kernel(in_features, W1, b1, W2, b2, neighbor_idx, rowptr):
    x = in_features[0]
    xp = jnp.pad(x, ((0, NPAD - N), (0, 0)))
    wc = jnp.concatenate([W1[:C_IN], W1[C_IN:]], axis=1)
    b1r = b1.reshape(1, HID)

    nblk = NPAD // 512
    p_arr, s_arr = pl.pallas_call(
        _mm_ps_body,
        grid=(nblk,),
        in_specs=[
            pl.BlockSpec((512, C_IN), lambda i: (i, 0)),
            pl.BlockSpec((C_IN, 2 * HID), lambda i: (0, 0)),
            pl.BlockSpec((1, HID), lambda i: (0, 0)),
        ],
        out_specs=[
            pl.BlockSpec((512, HID), lambda i: (i, 0)),
            pl.BlockSpec((512, HID), lambda i: (i, 0)),
        ],
        out_shape=[
            jax.ShapeDtypeStruct((NPAD, HID), jnp.float32),
            jax.ShapeDtypeStruct((NPAD, HID), jnp.float32),
        ],
    )(xp, wc, b1r)

    idx32 = neighbor_idx.astype(jnp.int32)
    rpt32 = rowptr.astype(jnp.int32)
    idxp = jnp.pad(idx32, (0, EPAD - E))
    rptp = jnp.pad(rpt32, (0, RPT_PAD - (N + 1)), constant_values=E)

    g_arr = _sc_kernel(p_arr, s_arr, idxp, rptp)

    rhi = rptp[1:NPAD + 1].reshape(nblk, 1, 512)
    rlo = rptp[:NPAD].reshape(nblk, 1, 512)
    b2r = b2.reshape(1, C_OUT)

    out = pl.pallas_call(
        _mm_out_body,
        grid=(nblk,),
        in_specs=[
            pl.BlockSpec((512, HID), lambda i: (i, 0)),
            pl.BlockSpec((HID, C_OUT), lambda i: (0, 0)),
            pl.BlockSpec((1, C_OUT), lambda i: (0, 0)),
            pl.BlockSpec((1, 1, 512), lambda i: (i, 0, 0)),
            pl.BlockSpec((1, 1, 512), lambda i: (i, 0, 0)),
        ],
        out_specs=pl.BlockSpec((512, C_OUT), lambda i: (i, 0)),
        out_shape=jax.ShapeDtypeStruct((NPAD, C_OUT), jnp.float32),
    )(g_arr, W2, b2r, rhi, rlo)

    return out[:N].reshape(1, N, C_OUT)


# speedup vs baseline: 8.6907x; 8.6907x over previous
import functools

import jax
import jax.numpy as jnp
from jax import lax
from jax.experimental import pallas as pl
from jax.experimental.pallas import tpu as pltpu
from jax.experimental.pallas import tpu_sc as plsc

N = 10000
E = 320000
C_IN = 128
HID = 256
C_OUT = 128

NC = 2
NS = 16
NW = NC * NS

NPW = 320
NPAD = NW * NPW
RPT_LEN = NPAD + 24
RPT_PAD = RPT_LEN
EW = E // NW
CH = 64
FB = 16
EPAD = E + CH

GA = -2.0 * 0.7978845608028654
GB = GA * 0.044715

VB = HID // 16


def _mm_ps_body(x_ref, w_ref, b1_ref, p_ref, s_ref):
    ps = jnp.dot(x_ref[...], w_ref[...], preferred_element_type=jnp.float32)
    p_ref[...] = ps[:, :HID]
    s_ref[...] = ps[:, HID:] + b1_ref[...]


def _mm_out_body(g_ref, w2_ref, b2_ref, rhi_ref, rlo_ref, o_ref):
    cnt = (rhi_ref[0, 0, :] - rlo_ref[0, 0, :]).astype(jnp.float32)
    scale = 1.0 / jnp.maximum(cnt, 1.0)
    gs = g_ref[...] * scale[:, None]
    y = jnp.dot(gs, w2_ref[...], preferred_element_type=jnp.float32)
    o_ref[...] = y + b2_ref[...] * (cnt > 0.0).astype(jnp.float32)[:, None]


def _sc_segment_gelu(p_hbm, s_hbm, idx_hbm, rpt_hbm, g_hbm,
                     rpt_v, idx2_v, rows2_v, s_grp, flush_v,
                     semi, semg, sem2):
    c = lax.axis_index("c")
    s = lax.axis_index("s")
    wid = s * NC + c

    pltpu.async_copy(rpt_hbm.at[pl.ds(0, RPT_LEN)], rpt_v, sem2).wait()

    def rv(k):
        return rpt_v[pl.ds(k, 16)][0]

    def split_at(t):
        target = t * EW

        def bs_body(_, st):
            base, n = st
            half = n // 2
            go_right = rv(base + half) < target
            base = jnp.where(go_right, base + half + 1, base)
            n = jnp.where(go_right, n - half - 1, half)
            return (base, n)

        base, _ = lax.fori_loop(0, 14, bs_body, (0, NPAD + 1))
        r = jnp.minimum(((base + 8) // 16) * 16, NPAD)
        return jnp.where(t >= NW, NPAD, r)

    n0 = split_at(wid)
    n1 = split_at(wid + 1)

    zeros16 = jnp.zeros((16,), jnp.float32)

    def issue_idx(a, slot):
        pltpu.async_copy(
            idx_hbm.at[pl.ds(pl.multiple_of(jnp.minimum(a, E), 8), CH)],
            idx2_v.at[slot], semi)

    def issue_gather(slot):
        pltpu.async_copy(p_hbm.at[idx2_v.at[slot]], rows2_v.at[slot], semg)

    def drain_gather():
        pltpu.make_async_copy(p_hbm.at[pl.ds(0, CH)], rows2_v.at[0],
                              semg).wait()

    def drain_idx():
        pltpu.make_async_copy(idx_hbm.at[pl.ds(0, CH)], idx2_v.at[0],
                              semi).wait()

    e_start = rv(n0)
    e_end = rv(n1)
    a_init = (e_start // 8) * 8
    nchunks = (e_end - a_init + CH - 1) // CH
    n_events = (n1 - n0) + jnp.maximum(nchunks - 1, 0)

    issue_idx(a_init, 0)
    drain_idx()
    issue_gather(0)
    issue_idx(a_init + CH, 1)
    drain_gather()
    drain_idx()
    issue_gather(1)
    issue_idx(a_init + 2 * CH, 0)

    def event_body(_, st):
        i, a, p, par, acc = st
        e0 = rv(i)
        e1 = rv(i + 1)
        chunk_end = a + CH
        hi = jnp.minimum(e1, chunk_end)
        sl = i % FB

        @pl.when(jnp.logical_and(sl == 0, p == e0))
        def _():
            pltpu.async_copy(
                s_hbm.at[pl.ds(pl.multiple_of(i, 8), FB)], s_grp,
                sem2).wait()

        srow = tuple(s_grp[sl, pl.ds(j * 16, 16)] for j in range(VB))

        def edge_body(r, acc_):
            new_acc = []
            for j in range(VB):
                x = rows2_v[par, r, pl.ds(j * 16, 16)] + srow[j]
                z = x * (GA + GB * (x * x))
                new_acc.append(acc_[j] + x / (1.0 + jnp.exp(z)))
            return tuple(new_acc)

        acc = lax.fori_loop(p - a, hi - a, edge_body, acc)
        node_done = jnp.logical_and(e1 <= chunk_end, i < n1)

        @pl.when(node_done)
        def _():
            for j in range(VB):
                flush_v[sl, pl.ds(j * 16, 16)] = acc[j]

        @pl.when(jnp.logical_and(node_done, sl == FB - 1))
        def _():
            pltpu.async_copy(
                flush_v,
                g_hbm.at[pl.ds(pl.multiple_of(i - (FB - 1), 8), FB)],
                sem2).wait()

        npar = 1 - par

        @pl.when(jnp.logical_not(node_done))
        def _():
            drain_gather()
            drain_idx()
            issue_gather(par)
            issue_idx(a + 3 * CH, npar)

        keep = jnp.where(node_done, 0.0, 1.0).astype(jnp.float32)
        acc = tuple(acc[j] * keep for j in range(VB))
        i = i + node_done.astype(jnp.int32)
        a = jnp.where(node_done, a, a + CH)
        par = jnp.where(node_done, par, npar)
        return (i, a, hi, par, acc)

    lax.fori_loop(0, n_events, event_body,
                  (n0, a_init, e_start, 0, (zeros16,) * VB))

    drain_gather()
    drain_idx()


@functools.partial(
    pl.kernel,
    mesh=plsc.VectorSubcoreMesh(core_axis_name="c", subcore_axis_name="s"),
    out_type=jax.ShapeDtypeStruct((NPAD, HID), jnp.float32),
    scratch_types=[
        pltpu.VMEM((RPT_LEN,), jnp.int32),
        pltpu.VMEM((2, CH), jnp.int32),
        pltpu.VMEM((2, CH, HID), jnp.float32),
        pltpu.VMEM((FB, HID), jnp.float32),
        pltpu.VMEM((FB, HID), jnp.float32),
        pltpu.SemaphoreType.DMA,
        pltpu.SemaphoreType.DMA,
        pltpu.SemaphoreType.DMA,
    ],
)
def _sc_kernel(p_hbm, s_hbm, idx_hbm, rpt_hbm, g_hbm,
               rpt_v, idx2_v, rows2_v, s_grp, flush_v, semi, semg, sem2):
    _sc_segment_gelu(p_hbm, s_hbm, idx_hbm, rpt_hbm, g_hbm,
                     rpt_v, idx2_v, rows2_v, s_grp, flush_v,
                     semi, semg, sem2)


def kernel(in_features, W1, b1, W2, b2, neighbor_idx, rowptr):
    x = in_features[0]
    xp = jnp.pad(x, ((0, NPAD - N), (0, 0)))
    wc = jnp.concatenate([W1[:C_IN], W1[C_IN:]], axis=1)
    b1r = b1.reshape(1, HID)

    nblk = NPAD // 512
    p_arr, s_arr = pl.pallas_call(
        _mm_ps_body,
        grid=(nblk,),
        in_specs=[
            pl.BlockSpec((512, C_IN), lambda i: (i, 0)),
            pl.BlockSpec((C_IN, 2 * HID), lambda i: (0, 0)),
            pl.BlockSpec((1, HID), lambda i: (0, 0)),
        ],
        out_specs=[
            pl.BlockSpec((512, HID), lambda i: (i, 0)),
            pl.BlockSpec((512, HID), lambda i: (i, 0)),
        ],
        out_shape=[
            jax.ShapeDtypeStruct((NPAD, HID), jnp.float32),
            jax.ShapeDtypeStruct((NPAD, HID), jnp.float32),
        ],
    )(xp, wc, b1r)

    idx32 = neighbor_idx.astype(jnp.int32)
    rpt32 = rowptr.astype(jnp.int32)
    idxp = jnp.pad(idx32, (0, EPAD - E))
    rptp = jnp.pad(rpt32, (0, RPT_PAD - (N + 1)), constant_values=E)

    g_arr = _sc_kernel(p_arr, s_arr, idxp, rptp)

    rhi = rptp[1:NPAD + 1].reshape(nblk, 1, 512)
    rlo = rptp[:NPAD].reshape(nblk, 1, 512)
    b2r = b2.reshape(1, C_OUT)

    out = pl.pallas_call(
        _mm_out_body,
        grid=(nblk,),
        in_specs=[
            pl.BlockSpec((512, HID), lambda i: (i, 0)),
            pl.BlockSpec((HID, C_OUT), lambda i: (0, 0)),
            pl.BlockSpec((1, C_OUT), lambda i: (0, 0)),
            pl.BlockSpec((1, 1, 512), lambda i: (i, 0, 0)),
            pl.BlockSpec((1, 1, 512), lambda i: (i, 0, 0)),
        ],
        out_specs=pl.BlockSpec((512, C_OUT), lambda i: (i, 0)),
        out_shape=jax.ShapeDtypeStruct((NPAD, C_OUT), jnp.float32),
    )(g_arr, W2, b2r, rhi, rlo)

    return out[:N].reshape(1, N, C_OUT)


# speedup vs baseline: 8.7441x; 1.0061x over previous
import functools

import jax
import jax.numpy as jnp
from jax import lax
from jax.experimental import pallas as pl
from jax.experimental.pallas import tpu as pltpu
from jax.experimental.pallas import tpu_sc as plsc

N = 10000
E = 320000
C_IN = 128
HID = 256
C_OUT = 128

NC = 2
NS = 16
NW = NC * NS

NPW = 320
NPAD = NW * NPW
RPT_LEN = NPAD + 24
RPT_PAD = RPT_LEN
EW = E // NW
CH = 128
FB = 16
EPAD = E + CH

GA = -2.0 * 0.7978845608028654
GB = GA * 0.044715

VB = HID // 16


def _mm_ps_body(x_ref, w_ref, b1_ref, p_ref, s_ref):
    ps = jnp.dot(x_ref[...], w_ref[...], preferred_element_type=jnp.float32)
    p_ref[...] = ps[:, :HID]
    s_ref[...] = ps[:, HID:] + b1_ref[...]


def _mm_out_body(g_ref, w2_ref, b2_ref, rhi_ref, rlo_ref, o_ref):
    cnt = (rhi_ref[0, 0, :] - rlo_ref[0, 0, :]).astype(jnp.float32)
    scale = 1.0 / jnp.maximum(cnt, 1.0)
    gs = g_ref[...] * scale[:, None]
    y = jnp.dot(gs, w2_ref[...], preferred_element_type=jnp.float32)
    o_ref[...] = y + b2_ref[...] * (cnt > 0.0).astype(jnp.float32)[:, None]


def _sc_segment_gelu(p_hbm, s_hbm, idx_hbm, rpt_hbm, g_hbm,
                     rpt_v, idx2_v, rows2_v, s_grp, flush_v,
                     semi, semg, sem2):
    c = lax.axis_index("c")
    s = lax.axis_index("s")
    wid = s * NC + c

    pltpu.async_copy(rpt_hbm.at[pl.ds(0, RPT_LEN)], rpt_v, sem2).wait()

    def rv(k):
        return rpt_v[pl.ds(k, 16)][0]

    def split_at(t):
        target = t * EW

        def bs_body(_, st):
            base, n = st
            half = n // 2
            go_right = rv(base + half) < target
            base = jnp.where(go_right, base + half + 1, base)
            n = jnp.where(go_right, n - half - 1, half)
            return (base, n)

        base, _ = lax.fori_loop(0, 14, bs_body, (0, NPAD + 1))
        r = jnp.minimum(((base + 8) // 16) * 16, NPAD)
        return jnp.where(t >= NW, NPAD, r)

    n0 = split_at(wid)
    n1 = split_at(wid + 1)

    zeros16 = jnp.zeros((16,), jnp.float32)

    def issue_idx(a, slot):
        pltpu.async_copy(
            idx_hbm.at[pl.ds(pl.multiple_of(jnp.minimum(a, E), 8), CH)],
            idx2_v.at[slot], semi)

    def issue_gather(slot):
        pltpu.async_copy(p_hbm.at[idx2_v.at[slot]], rows2_v.at[slot], semg)

    def drain_gather():
        pltpu.make_async_copy(p_hbm.at[pl.ds(0, CH)], rows2_v.at[0],
                              semg).wait()

    def drain_idx():
        pltpu.make_async_copy(idx_hbm.at[pl.ds(0, CH)], idx2_v.at[0],
                              semi).wait()

    e_start = rv(n0)
    e_end = rv(n1)
    a_init = (e_start // 8) * 8
    nchunks = (e_end - a_init + CH - 1) // CH
    n_events = (n1 - n0) + jnp.maximum(nchunks - 1, 0)

    issue_idx(a_init, 0)
    drain_idx()
    issue_gather(0)
    issue_idx(a_init + CH, 1)
    drain_gather()
    drain_idx()
    issue_gather(1)
    issue_idx(a_init + 2 * CH, 0)

    def event_body(_, st):
        i, a, p, par, acc = st
        e0 = rv(i)
        e1 = rv(i + 1)
        chunk_end = a + CH
        hi = jnp.minimum(e1, chunk_end)
        sl = i % FB

        @pl.when(jnp.logical_and(sl == 0, p == e0))
        def _():
            pltpu.async_copy(
                s_hbm.at[pl.ds(pl.multiple_of(i, 8), FB)], s_grp,
                sem2).wait()

        srow = tuple(s_grp[sl, pl.ds(j * 16, 16)] for j in range(VB))

        def edge_body(r, acc_):
            new_acc = []
            for j in range(VB):
                x = rows2_v[par, r, pl.ds(j * 16, 16)] + srow[j]
                z = x * (GA + GB * (x * x))
                new_acc.append(acc_[j] + x / (1.0 + jnp.exp(z)))
            return tuple(new_acc)

        acc = lax.fori_loop(p - a, hi - a, edge_body, acc)
        node_done = jnp.logical_and(e1 <= chunk_end, i < n1)

        @pl.when(node_done)
        def _():
            for j in range(VB):
                flush_v[sl, pl.ds(j * 16, 16)] = acc[j]

        @pl.when(jnp.logical_and(node_done, sl == FB - 1))
        def _():
            pltpu.async_copy(
                flush_v,
                g_hbm.at[pl.ds(pl.multiple_of(i - (FB - 1), 8), FB)],
                sem2).wait()

        npar = 1 - par

        @pl.when(jnp.logical_not(node_done))
        def _():
            drain_gather()
            drain_idx()
            issue_gather(par)
            issue_idx(a + 3 * CH, npar)

        keep = jnp.where(node_done, 0.0, 1.0).astype(jnp.float32)
        acc = tuple(acc[j] * keep for j in range(VB))
        i = i + node_done.astype(jnp.int32)
        a = jnp.where(node_done, a, a + CH)
        par = jnp.where(node_done, par, npar)
        return (i, a, hi, par, acc)

    lax.fori_loop(0, n_events, event_body,
                  (n0, a_init, e_start, 0, (zeros16,) * VB))

    drain_gather()
    drain_idx()


@functools.partial(
    pl.kernel,
    mesh=plsc.VectorSubcoreMesh(core_axis_name="c", subcore_axis_name="s"),
    out_type=jax.ShapeDtypeStruct((NPAD, HID), jnp.float32),
    scratch_types=[
        pltpu.VMEM((RPT_LEN,), jnp.int32),
        pltpu.VMEM((2, CH), jnp.int32),
        pltpu.VMEM((2, CH, HID), jnp.float32),
        pltpu.VMEM((FB, HID), jnp.float32),
        pltpu.VMEM((FB, HID), jnp.float32),
        pltpu.SemaphoreType.DMA,
        pltpu.SemaphoreType.DMA,
        pltpu.SemaphoreType.DMA,
    ],
)
def _sc_kernel(p_hbm, s_hbm, idx_hbm, rpt_hbm, g_hbm,
               rpt_v, idx2_v, rows2_v, s_grp, flush_v, semi, semg, sem2):
    _sc_segment_gelu(p_hbm, s_hbm, idx_hbm, rpt_hbm, g_hbm,
                     rpt_v, idx2_v, rows2_v, s_grp, flush_v,
                     semi, semg, sem2)


def kernel(in_features, W1, b1, W2, b2, neighbor_idx, rowptr):
    x = in_features[0]
    xp = jnp.pad(x, ((0, NPAD - N), (0, 0)))
    wc = jnp.concatenate([W1[:C_IN], W1[C_IN:]], axis=1)
    b1r = b1.reshape(1, HID)

    nblk = NPAD // 512
    p_arr, s_arr = pl.pallas_call(
        _mm_ps_body,
        grid=(nblk,),
        in_specs=[
            pl.BlockSpec((512, C_IN), lambda i: (i, 0)),
            pl.BlockSpec((C_IN, 2 * HID), lambda i: (0, 0)),
            pl.BlockSpec((1, HID), lambda i: (0, 0)),
        ],
        out_specs=[
            pl.BlockSpec((512, HID), lambda i: (i, 0)),
            pl.BlockSpec((512, HID), lambda i: (i, 0)),
        ],
        out_shape=[
            jax.ShapeDtypeStruct((NPAD, HID), jnp.float32),
            jax.ShapeDtypeStruct((NPAD, HID), jnp.float32),
        ],
    )(xp, wc, b1r)

    idx32 = neighbor_idx.astype(jnp.int32)
    rpt32 = rowptr.astype(jnp.int32)
    idxp = jnp.pad(idx32, (0, EPAD - E))
    rptp = jnp.pad(rpt32, (0, RPT_PAD - (N + 1)), constant_values=E)

    g_arr = _sc_kernel(p_arr, s_arr, idxp, rptp)

    rhi = rptp[1:NPAD + 1].reshape(nblk, 1, 512)
    rlo = rptp[:NPAD].reshape(nblk, 1, 512)
    b2r = b2.reshape(1, C_OUT)

    out = pl.pallas_call(
        _mm_out_body,
        grid=(nblk,),
        in_specs=[
            pl.BlockSpec((512, HID), lambda i: (i, 0)),
            pl.BlockSpec((HID, C_OUT), lambda i: (0, 0)),
            pl.BlockSpec((1, C_OUT), lambda i: (0, 0)),
            pl.BlockSpec((1, 1, 512), lambda i: (i, 0, 0)),
            pl.BlockSpec((1, 1, 512), lambda i: (i, 0, 0)),
        ],
        out_specs=pl.BlockSpec((512, C_OUT), lambda i: (i, 0)),
        out_shape=jax.ShapeDtypeStruct((NPAD, C_OUT), jnp.float32),
    )(g_arr, W2, b2r, rhi, rlo)

    return out[:N].reshape(1, N, C_OUT)
